# Initial kernel scaffold; baseline (speedup 1.0000x reference)
import jax
import jax.numpy as jnp
from jax.experimental import pallas as pl


def kernel(X, edge_index, edge_weight, H, C, W_i, b_i, W_f, b_f, W_c, b_c, W_o, b_o, Wc_i, bc_i, Wc_f, bc_f, Wc_c, bc_c, Wc_o, bc_o):
    raise NotImplementedError("write your pallas kernel here")


# speedup vs baseline: 4.3552x; 4.3552x over previous
import functools

import jax
import jax.numpy as jnp
from jax import lax
from jax.experimental import pallas as pl
from jax.experimental.pallas import tpu as pltpu
from jax.experimental.pallas import tpu_sc as plsc

N = 10000
E = 160000
D = 256
NC, NS, L = 2, 16, 16
HC = D // NC
NPH = 5
NBP = 16
EB = 128
EP = 163840
ER = EP // EB
NB_ALL = EP // NS // EB
NB_HALF = EP // (NC * NS) // EB
NP = 10240
NPAD = 10240
ROWS_T = NPAD // NS
RB = 128

_mesh = plsc.VectorSubcoreMesh(
    core_axis_name="c", subcore_axis_name="s", num_cores=NC, num_subcores=NS)
_sc_params = pltpu.CompilerParams(needs_layout_passes=False)


@functools.partial(
    pl.kernel,
    out_type=jax.ShapeDtypeStruct((NC, NP), jnp.float32),
    mesh=_mesh,
    scratch_types=[
        pltpu.VMEM((NB_HALF, EB), jnp.int32),
        pltpu.VMEM((NB_HALF, EB), jnp.float32),
        pltpu.VMEM((NP // NS,), jnp.float32),
        pltpu.VMEM_SHARED((NP,), jnp.float32),
    ],
)
def _deg_call(src_hbm, ew_hbm, out_hbm, src_v, ew_v, stage_v, acc_sh):
    c = lax.axis_index("c")
    s = lax.axis_index("s")
    wid = c * NS + s
    seg = NP // NS

    @pl.loop(0, seg // L)
    def _zero(i):
        stage_v[pl.ds(i * L, L)] = jnp.zeros((L,), jnp.float32)

    pltpu.sync_copy(stage_v, acc_sh.at[pl.ds(s * seg, seg)])
    plsc.subcore_barrier()

    r0 = wid * NB_HALF
    pltpu.sync_copy(src_hbm.at[pl.ds(r0, NB_HALF)], src_v)
    pltpu.sync_copy(ew_hbm.at[pl.ds(r0, NB_HALF)], ew_v)

    @pl.loop(0, NB_HALF)
    def _scat(b):
        pltpu.sync_copy(ew_v.at[b], acc_sh.at[src_v.at[b]], add=True)

    plsc.subcore_barrier()
    pltpu.sync_copy(acc_sh.at[pl.ds(s * seg, seg)], stage_v)
    pltpu.sync_copy(stage_v, out_hbm.at[c, pl.ds(s * seg, seg)])


def _dis_body(deg_ref, dis_ref):
    d = deg_ref[0] + deg_ref[1]
    dis_ref[...] = jnp.where(d > 0.0, lax.rsqrt(d), 0.0)


_dis_call = pl.pallas_call(
    _dis_body,
    out_shape=jax.ShapeDtypeStruct((NP // 128, 128), jnp.float32),
)


@functools.partial(
    pl.kernel,
    out_type=jax.ShapeDtypeStruct((ER, EB), jnp.float32),
    mesh=_mesh,
    scratch_types=[
        pltpu.VMEM((NP,), jnp.float32),
        pltpu.VMEM((NB_HALF, EB), jnp.int32),
        pltpu.VMEM((NB_HALF, EB), jnp.int32),
        pltpu.VMEM((NB_HALF, EB), jnp.float32),
        pltpu.VMEM((NB_HALF, EB), jnp.float32),
    ],
    compiler_params=_sc_params,
)
def _norm_call(src_hbm, dst_hbm, ew_hbm, dis_hbm, out_hbm,
               dis_v, src_v, dst_v, ew_v, nrm_v):
    c = lax.axis_index("c")
    s = lax.axis_index("s")
    wid = c * NS + s
    pltpu.sync_copy(dis_hbm, dis_v)
    r0 = wid * NB_HALF
    pltpu.sync_copy(src_hbm.at[pl.ds(r0, NB_HALF)], src_v)
    pltpu.sync_copy(dst_hbm.at[pl.ds(r0, NB_HALF)], dst_v)
    pltpu.sync_copy(ew_hbm.at[pl.ds(r0, NB_HALF)], ew_v)

    @pl.loop(0, NB_HALF)
    def _row(b):
        for j in range(EB // L):
            sl = pl.ds(j * L, L)
            gs = plsc.load_gather(dis_v, [src_v[b, sl]])
            gd = plsc.load_gather(dis_v, [dst_v[b, sl]])
            nrm_v[b, sl] = -(ew_v[b, sl] * gs * gd)

    pltpu.sync_copy(nrm_v, out_hbm.at[pl.ds(r0, NB_HALF)])


def _make_mv(with_sub):
    scratch = [
        pltpu.VMEM((NBP, EB), jnp.int32),
        pltpu.VMEM((NBP, EB), jnp.int32),
        pltpu.VMEM((NBP, EB), jnp.float32),
        pltpu.VMEM((2, EB, HC), jnp.float32),
        pltpu.VMEM_SHARED((NPAD, HC), jnp.float32),
    ] + [pltpu.SemaphoreType.DMA] * 4

    def body(v_hbm, src_hbm, dst_hbm, nrm_hbm, *rest):
        if with_sub:
            sub_hbm, out_hbm = rest[0], rest[1]
            scr = rest[2:]
        else:
            sub_hbm, out_hbm = None, rest[0]
            scr = rest[1:]
        (src_v, dst_v, nrm_v, rows_v, acc_sh, g0, g1, s0, s1) = scr
        gsem = (g0, g1)
        ssem = (s0, s1)

        c = lax.axis_index("c")
        s = lax.axis_index("s")
        base = s * ROWS_T
        vch = v_hbm.at[c]

        @pl.loop(0, RB)
        def _zr(r):
            for j in range(HC // L):
                rows_v[0, r, pl.ds(j * L, L)] = jnp.zeros((L,), jnp.float32)

        for k in range(5):
            pltpu.sync_copy(rows_v.at[0], acc_sh.at[pl.ds(base + k * RB, RB)])
        plsc.subcore_barrier()

        for ph in range(NPH):
            r0 = s * NB_ALL + ph * NBP
            pltpu.sync_copy(src_hbm.at[pl.ds(r0, NBP)], src_v)
            pltpu.sync_copy(dst_hbm.at[pl.ds(r0, NBP)], dst_v)
            pltpu.sync_copy(nrm_hbm.at[pl.ds(r0, NBP)], nrm_v)

            pltpu.async_copy(vch.at[src_v.at[0]], rows_v.at[0], gsem[0])

            @pl.loop(0, NBP // 2)
            def _grp(g):
                for p in range(2):
                    b = g * 2 + p

                    pltpu.make_async_copy(
                        vch.at[src_v.at[0]], rows_v.at[p], gsem[p]).wait()

                    @pl.loop(0, EB // L)
                    def _sc(i16):
                        nv = nrm_v[b, pl.ds(i16 * L, L)]
                        for ii in range(L):
                            t = nv[ii]
                            i = i16 * L + ii
                            for j in range(HC // L):
                                sl = pl.ds(j * L, L)
                                rows_v[p, i, sl] = rows_v[p, i, sl] * t

                    @pl.when(b >= 1)
                    def _sw():
                        pltpu.make_async_copy(
                            rows_v.at[1 - p], acc_sh.at[dst_v.at[0]],
                            ssem[1 - p]).wait()

                    pltpu.async_copy(
                        rows_v.at[p], acc_sh.at[dst_v.at[b]], ssem[p], add=True)

                    @pl.when(b + 1 < NBP)
                    def _gs():
                        pltpu.async_copy(
                            vch.at[src_v.at[b + 1]], rows_v.at[1 - p],
                            gsem[1 - p])

            pltpu.make_async_copy(
                rows_v.at[1], acc_sh.at[dst_v.at[0]], ssem[1]).wait()

        plsc.subcore_barrier()

        for k in range(5):
            rr = base + k * RB
            if with_sub:
                pltpu.sync_copy(acc_sh.at[pl.ds(rr, RB)], rows_v.at[0])
                pltpu.sync_copy(sub_hbm.at[c, pl.ds(rr, RB)], rows_v.at[1])

                @pl.loop(0, RB)
                def _cmb(r):
                    for j in range(HC // L):
                        sl = pl.ds(j * L, L)
                        rows_v[0, r, sl] = (rows_v[0, r, sl] * 2.0
                                            - rows_v[1, r, sl])

                pltpu.sync_copy(rows_v.at[0], out_hbm.at[c, pl.ds(rr, RB)])
            else:
                pltpu.sync_copy(acc_sh.at[pl.ds(rr, RB)],
                                out_hbm.at[c, pl.ds(rr, RB)])

    return functools.partial(
        pl.kernel,
        out_type=jax.ShapeDtypeStruct((NC, NPAD, HC), jnp.float32),
        mesh=_mesh,
        scratch_types=scratch,
        compiler_params=_sc_params,
    )(body)


_mv_plain = _make_mv(False)
_mv_sub = _make_mv(True)


RBLK = 1000


def _gate_body(x_ref, h_ref, c_ref, t1_ref, t2_ref, t3_ref, w_ref, b_ref,
               hn_ref, cn_ref):
    f32 = jnp.float32
    acc = jnp.dot(x_ref[...], w_ref[0:256, :], preferred_element_type=f32)
    acc = acc + jnp.dot(h_ref[...], w_ref[256:512, :], preferred_element_type=f32)
    off = 512
    for t_ref in (t1_ref, t2_ref, t3_ref):
        t = jnp.concatenate([t_ref[0], t_ref[1]], axis=1)
        acc = acc + jnp.dot(t, w_ref[off:off + 256, :],
                            preferred_element_type=f32)
        off += 256
    acc = acc + b_ref[...]
    gi = jax.nn.sigmoid(acc[:, 0:256])
    gf = jax.nn.sigmoid(acc[:, 256:512])
    gt = jnp.tanh(acc[:, 512:768])
    go = jax.nn.sigmoid(acc[:, 768:1024])
    cn = gf * c_ref[...] + gi * gt
    cn_ref[...] = cn
    hn_ref[...] = go * jnp.tanh(cn)


_gate_call = pl.pallas_call(
    _gate_body,
    grid=(N // RBLK,),
    in_specs=[
        pl.BlockSpec((RBLK, D), lambda i: (i, 0)),
        pl.BlockSpec((RBLK, D), lambda i: (i, 0)),
        pl.BlockSpec((RBLK, D), lambda i: (i, 0)),
        pl.BlockSpec((NC, RBLK, HC), lambda i: (0, i, 0)),
        pl.BlockSpec((NC, RBLK, HC), lambda i: (0, i, 0)),
        pl.BlockSpec((NC, RBLK, HC), lambda i: (0, i, 0)),
        pl.BlockSpec((5 * D, 4 * D), lambda i: (0, 0)),
        pl.BlockSpec((1, 4 * D), lambda i: (0, 0)),
    ],
    out_specs=[
        pl.BlockSpec((RBLK, D), lambda i: (i, 0)),
        pl.BlockSpec((RBLK, D), lambda i: (i, 0)),
    ],
    out_shape=[
        jax.ShapeDtypeStruct((N, D), jnp.float32),
        jax.ShapeDtypeStruct((N, D), jnp.float32),
    ],
)


def kernel(X, edge_index, edge_weight, H, C,
           W_i, b_i, W_f, b_f, W_c, b_c, W_o, b_o,
           Wc_i, bc_i, Wc_f, bc_f, Wc_c, bc_c, Wc_o, bc_o):
    f32 = jnp.float32
    src = edge_index[0]
    dst = edge_index[1]
    pad = EP - E
    srcp = jnp.concatenate([src, jnp.zeros((pad,), jnp.int32)]).reshape(ER, EB)
    dstp = jnp.concatenate([dst, jnp.zeros((pad,), jnp.int32)]).reshape(ER, EB)
    ewp = jnp.concatenate([edge_weight, jnp.zeros((pad,), f32)]).reshape(ER, EB)

    deg2 = _deg_call(srcp, ewp)
    dis = _dis_call(deg2.reshape(NC, NP // 128, 128))
    nrm = _norm_call(srcp, dstp, ewp, dis.reshape(NP))

    Hpad = jnp.concatenate([H, jnp.zeros((NPAD - N, D), f32)])
    Hc = Hpad.reshape(NPAD, NC, HC).transpose(1, 0, 2)
    Tx1 = _mv_plain(Hc, srcp, dstp, nrm)
    Tx2 = _mv_sub(Tx1, srcp, dstp, nrm, Hc)
    Tx3 = _mv_sub(Tx2, srcp, dstp, nrm, Tx1)

    Wbig = jnp.concatenate([
        jnp.concatenate([W_i, W_f, W_c, W_o], axis=1),
        jnp.concatenate([Wc_i[0], Wc_f[0], Wc_c[0], Wc_o[0]], axis=1),
        jnp.concatenate([Wc_i[1], Wc_f[1], Wc_c[1], Wc_o[1]], axis=1),
        jnp.concatenate([Wc_i[2], Wc_f[2], Wc_c[2], Wc_o[2]], axis=1),
        jnp.concatenate([Wc_i[3], Wc_f[3], Wc_c[3], Wc_o[3]], axis=1),
    ], axis=0)
    bias = jnp.concatenate([
        b_i + bc_i[None, :], b_f + bc_f[None, :],
        b_c + bc_c[None, :], b_o + bc_o[None, :],
    ], axis=1)

    Hn, Cn = _gate_call(X, H, C, Tx1, Tx2, Tx3, Wbig, bias)
    return (Hn, Cn)


# speedup vs baseline: 4.8186x; 1.1064x over previous
import functools

import jax
import jax.numpy as jnp
from jax import lax
from jax.experimental import pallas as pl
from jax.experimental.pallas import tpu as pltpu
from jax.experimental.pallas import tpu_sc as plsc

N = 10000
E = 160000
D = 256
NC, NS, L = 2, 16, 16
HC = D // NC
NPH = 5
NBP = 16
EB = 128
EP = 163840
ER = EP // EB
NB_ALL = EP // NS // EB
NB_HALF = EP // (NC * NS) // EB
NP = 10240
NPAD = 10240
ROWS_T = NPAD // NS
RB = 128

_mesh = plsc.VectorSubcoreMesh(
    core_axis_name="c", subcore_axis_name="s", num_cores=NC, num_subcores=NS)
_sc_params = pltpu.CompilerParams(needs_layout_passes=False)


@functools.partial(
    pl.kernel,
    out_type=jax.ShapeDtypeStruct((NC, NP), jnp.float32),
    mesh=_mesh,
    scratch_types=[
        pltpu.VMEM((NB_HALF, EB), jnp.int32),
        pltpu.VMEM((NB_HALF, EB), jnp.float32),
        pltpu.VMEM((NP // NS,), jnp.float32),
        pltpu.VMEM_SHARED((NP,), jnp.float32),
    ],
)
def _deg_call(src_hbm, ew_hbm, out_hbm, src_v, ew_v, stage_v, acc_sh):
    c = lax.axis_index("c")
    s = lax.axis_index("s")
    wid = c * NS + s
    seg = NP // NS

    @pl.loop(0, seg // L)
    def _zero(i):
        stage_v[pl.ds(i * L, L)] = jnp.zeros((L,), jnp.float32)

    pltpu.sync_copy(stage_v, acc_sh.at[pl.ds(s * seg, seg)])
    plsc.subcore_barrier()

    r0 = wid * NB_HALF
    pltpu.sync_copy(src_hbm.at[pl.ds(r0, NB_HALF)], src_v)
    pltpu.sync_copy(ew_hbm.at[pl.ds(r0, NB_HALF)], ew_v)

    @pl.loop(0, NB_HALF)
    def _scat(b):
        pltpu.sync_copy(ew_v.at[b], acc_sh.at[src_v.at[b]], add=True)

    plsc.subcore_barrier()
    pltpu.sync_copy(acc_sh.at[pl.ds(s * seg, seg)], stage_v)
    pltpu.sync_copy(stage_v, out_hbm.at[c, pl.ds(s * seg, seg)])


def _dis_body(deg_ref, dis_ref):
    d = deg_ref[0] + deg_ref[1]
    dis_ref[...] = jnp.where(d > 0.0, lax.rsqrt(d), 0.0)


_dis_call = pl.pallas_call(
    _dis_body,
    out_shape=jax.ShapeDtypeStruct((NP // 128, 128), jnp.float32),
)


@functools.partial(
    pl.kernel,
    out_type=jax.ShapeDtypeStruct((ER, EB), jnp.float32),
    mesh=_mesh,
    scratch_types=[
        pltpu.VMEM((NP,), jnp.float32),
        pltpu.VMEM((NB_HALF, EB), jnp.int32),
        pltpu.VMEM((NB_HALF, EB), jnp.int32),
        pltpu.VMEM((NB_HALF, EB), jnp.float32),
        pltpu.VMEM((NB_HALF, EB), jnp.float32),
    ],
    compiler_params=_sc_params,
)
def _norm_call(src_hbm, dst_hbm, ew_hbm, dis_hbm, out_hbm,
               dis_v, src_v, dst_v, ew_v, nrm_v):
    c = lax.axis_index("c")
    s = lax.axis_index("s")
    wid = c * NS + s
    pltpu.sync_copy(dis_hbm, dis_v)
    r0 = wid * NB_HALF
    pltpu.sync_copy(src_hbm.at[pl.ds(r0, NB_HALF)], src_v)
    pltpu.sync_copy(dst_hbm.at[pl.ds(r0, NB_HALF)], dst_v)
    pltpu.sync_copy(ew_hbm.at[pl.ds(r0, NB_HALF)], ew_v)

    @pl.loop(0, NB_HALF)
    def _row(b):
        for j in range(EB // L):
            sl = pl.ds(j * L, L)
            gs = plsc.load_gather(dis_v, [src_v[b, sl]])
            gd = plsc.load_gather(dis_v, [dst_v[b, sl]])
            nrm_v[b, sl] = -(ew_v[b, sl] * gs * gd)

    pltpu.sync_copy(nrm_v, out_hbm.at[pl.ds(r0, NB_HALF)])


def _make_mv(with_sub):
    scratch = [
        pltpu.VMEM((NBP, EB), jnp.int32),
        pltpu.VMEM((NBP, EB), jnp.int32),
        pltpu.VMEM((NBP, EB), jnp.float32),
        pltpu.VMEM((2, EB, HC), jnp.float32),
        pltpu.VMEM_SHARED((NPAD, HC), jnp.float32),
    ] + [pltpu.SemaphoreType.DMA] * 4

    def body(v_hbm, src_hbm, dst_hbm, nrm_hbm, *rest):
        if with_sub:
            sub_hbm, out_hbm = rest[0], rest[1]
            scr = rest[2:]
        else:
            sub_hbm, out_hbm = None, rest[0]
            scr = rest[1:]
        (src_v, dst_v, nrm_v, rows_v, acc_sh, g0, g1, s0, s1) = scr
        gsem = (g0, g1)
        ssem = (s0, s1)

        c = lax.axis_index("c")
        s = lax.axis_index("s")
        base = s * ROWS_T
        vch = v_hbm.at[c]

        @pl.loop(0, RB)
        def _zr(r):
            for j in range(HC // L):
                rows_v[0, r, pl.ds(j * L, L)] = jnp.zeros((L,), jnp.float32)

        for k in range(5):
            pltpu.sync_copy(rows_v.at[0], acc_sh.at[pl.ds(base + k * RB, RB)])
        plsc.subcore_barrier()

        for ph in range(NPH):
            r0 = s * NB_ALL + ph * NBP
            pltpu.sync_copy(src_hbm.at[pl.ds(r0, NBP)], src_v)
            pltpu.sync_copy(dst_hbm.at[pl.ds(r0, NBP)], dst_v)
            pltpu.sync_copy(nrm_hbm.at[pl.ds(r0, NBP)], nrm_v)

            pltpu.async_copy(vch.at[src_v.at[0]], rows_v.at[0], gsem[0])

            @pl.loop(0, NBP // 2)
            def _grp(g):
                for p in range(2):
                    b = g * 2 + p

                    pltpu.make_async_copy(
                        vch.at[src_v.at[0]], rows_v.at[p], gsem[p]).wait()

                    @pl.when(b >= 1)
                    def _sw():
                        pltpu.make_async_copy(
                            rows_v.at[1 - p], acc_sh.at[dst_v.at[0]],
                            ssem[1 - p]).wait()

                    @pl.when(b + 1 < NBP)
                    def _gs():
                        pltpu.async_copy(
                            vch.at[src_v.at[b + 1]], rows_v.at[1 - p],
                            gsem[1 - p])

                    @pl.loop(0, EB // L)
                    def _sc(i16):
                        nv = nrm_v[b, pl.ds(i16 * L, L)]
                        for ii in range(L):
                            t = nv[ii]
                            i = i16 * L + ii
                            for j in range(HC // L):
                                sl = pl.ds(j * L, L)
                                rows_v[p, i, sl] = rows_v[p, i, sl] * t

                    pltpu.async_copy(
                        rows_v.at[p], acc_sh.at[dst_v.at[b]], ssem[p], add=True)

            pltpu.make_async_copy(
                rows_v.at[1], acc_sh.at[dst_v.at[0]], ssem[1]).wait()

        plsc.subcore_barrier()

        for k in range(5):
            rr = base + k * RB
            if with_sub:
                pltpu.sync_copy(acc_sh.at[pl.ds(rr, RB)], rows_v.at[0])
                pltpu.sync_copy(sub_hbm.at[c, pl.ds(rr, RB)], rows_v.at[1])

                @pl.loop(0, RB)
                def _cmb(r):
                    for j in range(HC // L):
                        sl = pl.ds(j * L, L)
                        rows_v[0, r, sl] = (rows_v[0, r, sl] * 2.0
                                            - rows_v[1, r, sl])

                pltpu.sync_copy(rows_v.at[0], out_hbm.at[c, pl.ds(rr, RB)])
            else:
                pltpu.sync_copy(acc_sh.at[pl.ds(rr, RB)],
                                out_hbm.at[c, pl.ds(rr, RB)])

    return functools.partial(
        pl.kernel,
        out_type=jax.ShapeDtypeStruct((NC, NPAD, HC), jnp.float32),
        mesh=_mesh,
        scratch_types=scratch,
        compiler_params=_sc_params,
    )(body)


_mv_plain = _make_mv(False)
_mv_sub = _make_mv(True)


RBLK = 1000


def _gate_body(x_ref, h_ref, c_ref, t1_ref, t2_ref, t3_ref, w_ref, b_ref,
               hn_ref, cn_ref):
    f32 = jnp.float32
    acc = jnp.dot(x_ref[...], w_ref[0:256, :], preferred_element_type=f32)
    acc = acc + jnp.dot(h_ref[...], w_ref[256:512, :], preferred_element_type=f32)
    off = 512
    for t_ref in (t1_ref, t2_ref, t3_ref):
        t = jnp.concatenate([t_ref[0], t_ref[1]], axis=1)
        acc = acc + jnp.dot(t, w_ref[off:off + 256, :],
                            preferred_element_type=f32)
        off += 256
    acc = acc + b_ref[...]
    gi = jax.nn.sigmoid(acc[:, 0:256])
    gf = jax.nn.sigmoid(acc[:, 256:512])
    gt = jnp.tanh(acc[:, 512:768])
    go = jax.nn.sigmoid(acc[:, 768:1024])
    cn = gf * c_ref[...] + gi * gt
    cn_ref[...] = cn
    hn_ref[...] = go * jnp.tanh(cn)


_gate_call = pl.pallas_call(
    _gate_body,
    grid=(N // RBLK,),
    in_specs=[
        pl.BlockSpec((RBLK, D), lambda i: (i, 0)),
        pl.BlockSpec((RBLK, D), lambda i: (i, 0)),
        pl.BlockSpec((RBLK, D), lambda i: (i, 0)),
        pl.BlockSpec((NC, RBLK, HC), lambda i: (0, i, 0)),
        pl.BlockSpec((NC, RBLK, HC), lambda i: (0, i, 0)),
        pl.BlockSpec((NC, RBLK, HC), lambda i: (0, i, 0)),
        pl.BlockSpec((5 * D, 4 * D), lambda i: (0, 0)),
        pl.BlockSpec((1, 4 * D), lambda i: (0, 0)),
    ],
    out_specs=[
        pl.BlockSpec((RBLK, D), lambda i: (i, 0)),
        pl.BlockSpec((RBLK, D), lambda i: (i, 0)),
    ],
    out_shape=[
        jax.ShapeDtypeStruct((N, D), jnp.float32),
        jax.ShapeDtypeStruct((N, D), jnp.float32),
    ],
)


def kernel(X, edge_index, edge_weight, H, C,
           W_i, b_i, W_f, b_f, W_c, b_c, W_o, b_o,
           Wc_i, bc_i, Wc_f, bc_f, Wc_c, bc_c, Wc_o, bc_o):
    f32 = jnp.float32
    src = edge_index[0]
    dst = edge_index[1]
    pad = EP - E
    srcp = jnp.concatenate([src, jnp.zeros((pad,), jnp.int32)]).reshape(ER, EB)
    dstp = jnp.concatenate([dst, jnp.zeros((pad,), jnp.int32)]).reshape(ER, EB)
    ewp = jnp.concatenate([edge_weight, jnp.zeros((pad,), f32)]).reshape(ER, EB)

    deg2 = _deg_call(srcp, ewp)
    dis = _dis_call(deg2.reshape(NC, NP // 128, 128))
    nrm = _norm_call(srcp, dstp, ewp, dis.reshape(NP))

    Hpad = jnp.concatenate([H, jnp.zeros((NPAD - N, D), f32)])
    Hc = Hpad.reshape(NPAD, NC, HC).transpose(1, 0, 2)
    Tx1 = _mv_plain(Hc, srcp, dstp, nrm)
    Tx2 = _mv_sub(Tx1, srcp, dstp, nrm, Hc)
    Tx3 = _mv_sub(Tx2, srcp, dstp, nrm, Tx1)

    Wbig = jnp.concatenate([
        jnp.concatenate([W_i, W_f, W_c, W_o], axis=1),
        jnp.concatenate([Wc_i[0], Wc_f[0], Wc_c[0], Wc_o[0]], axis=1),
        jnp.concatenate([Wc_i[1], Wc_f[1], Wc_c[1], Wc_o[1]], axis=1),
        jnp.concatenate([Wc_i[2], Wc_f[2], Wc_c[2], Wc_o[2]], axis=1),
        jnp.concatenate([Wc_i[3], Wc_f[3], Wc_c[3], Wc_o[3]], axis=1),
    ], axis=0)
    bias = jnp.concatenate([
        b_i + bc_i[None, :], b_f + bc_f[None, :],
        b_c + bc_c[None, :], b_o + bc_o[None, :],
    ], axis=1)

    Hn, Cn = _gate_call(X, H, C, Tx1, Tx2, Tx3, Wbig, bias)
    return (Hn, Cn)


# speedup vs baseline: 4.9673x; 1.0309x over previous
import functools

import jax
import jax.numpy as jnp
from jax import lax
from jax.experimental import pallas as pl
from jax.experimental.pallas import tpu as pltpu
from jax.experimental.pallas import tpu_sc as plsc

N = 10000
E = 160000
D = 256
NC, NS, L = 2, 16, 16
HC = D // NC
NPH = 2
NBP = 40
EB = 128
EP = 163840
ER = EP // EB
NB_ALL = EP // NS // EB
NB_HALF = EP // (NC * NS) // EB
NP = 10240
NPAD = 10240
ROWS_T = NPAD // NS
RB = 128

_mesh = plsc.VectorSubcoreMesh(
    core_axis_name="c", subcore_axis_name="s", num_cores=NC, num_subcores=NS)
_sc_params = pltpu.CompilerParams(needs_layout_passes=False)


@functools.partial(
    pl.kernel,
    out_type=jax.ShapeDtypeStruct((NC, NP), jnp.float32),
    mesh=_mesh,
    scratch_types=[
        pltpu.VMEM((NB_HALF, EB), jnp.int32),
        pltpu.VMEM((NB_HALF, EB), jnp.float32),
        pltpu.VMEM((NP // NS,), jnp.float32),
        pltpu.VMEM_SHARED((NP,), jnp.float32),
    ],
)
def _deg_call(src_hbm, ew_hbm, out_hbm, src_v, ew_v, stage_v, acc_sh):
    c = lax.axis_index("c")
    s = lax.axis_index("s")
    wid = c * NS + s
    seg = NP // NS

    @pl.loop(0, seg // L)
    def _zero(i):
        stage_v[pl.ds(i * L, L)] = jnp.zeros((L,), jnp.float32)

    pltpu.sync_copy(stage_v, acc_sh.at[pl.ds(s * seg, seg)])
    plsc.subcore_barrier()

    r0 = wid * NB_HALF
    pltpu.sync_copy(src_hbm.at[pl.ds(r0, NB_HALF)], src_v)
    pltpu.sync_copy(ew_hbm.at[pl.ds(r0, NB_HALF)], ew_v)

    @pl.loop(0, NB_HALF)
    def _scat(b):
        pltpu.sync_copy(ew_v.at[b], acc_sh.at[src_v.at[b]], add=True)

    plsc.subcore_barrier()
    pltpu.sync_copy(acc_sh.at[pl.ds(s * seg, seg)], stage_v)
    pltpu.sync_copy(stage_v, out_hbm.at[c, pl.ds(s * seg, seg)])


def _dis_body(deg_ref, dis_ref):
    d = deg_ref[0] + deg_ref[1]
    dis_ref[...] = jnp.where(d > 0.0, lax.rsqrt(d), 0.0)


_dis_call = pl.pallas_call(
    _dis_body,
    out_shape=jax.ShapeDtypeStruct((NP // 128, 128), jnp.float32),
)


@functools.partial(
    pl.kernel,
    out_type=jax.ShapeDtypeStruct((ER, EB), jnp.float32),
    mesh=_mesh,
    scratch_types=[
        pltpu.VMEM((NP,), jnp.float32),
        pltpu.VMEM((NB_HALF, EB), jnp.int32),
        pltpu.VMEM((NB_HALF, EB), jnp.int32),
        pltpu.VMEM((NB_HALF, EB), jnp.float32),
        pltpu.VMEM((NB_HALF, EB), jnp.float32),
    ],
    compiler_params=_sc_params,
)
def _norm_call(src_hbm, dst_hbm, ew_hbm, dis_hbm, out_hbm,
               dis_v, src_v, dst_v, ew_v, nrm_v):
    c = lax.axis_index("c")
    s = lax.axis_index("s")
    wid = c * NS + s
    pltpu.sync_copy(dis_hbm, dis_v)
    r0 = wid * NB_HALF
    pltpu.sync_copy(src_hbm.at[pl.ds(r0, NB_HALF)], src_v)
    pltpu.sync_copy(dst_hbm.at[pl.ds(r0, NB_HALF)], dst_v)
    pltpu.sync_copy(ew_hbm.at[pl.ds(r0, NB_HALF)], ew_v)

    @pl.loop(0, NB_HALF)
    def _row(b):
        for j in range(EB // L):
            sl = pl.ds(j * L, L)
            gs = plsc.load_gather(dis_v, [src_v[b, sl]])
            gd = plsc.load_gather(dis_v, [dst_v[b, sl]])
            nrm_v[b, sl] = -(ew_v[b, sl] * gs * gd)

    pltpu.sync_copy(nrm_v, out_hbm.at[pl.ds(r0, NB_HALF)])


def _make_mv(with_sub):
    scratch = [
        pltpu.VMEM((NBP, EB), jnp.int32),
        pltpu.VMEM((NBP, EB), jnp.int32),
        pltpu.VMEM((NBP, EB), jnp.float32),
        pltpu.VMEM((2, EB, HC), jnp.float32),
        pltpu.VMEM_SHARED((NPAD, HC), jnp.float32),
    ] + [pltpu.SemaphoreType.DMA] * 4

    def body(v_hbm, src_hbm, dst_hbm, nrm_hbm, *rest):
        if with_sub:
            sub_hbm, out_hbm = rest[0], rest[1]
            scr = rest[2:]
        else:
            sub_hbm, out_hbm = None, rest[0]
            scr = rest[1:]
        (src_v, dst_v, nrm_v, rows_v, acc_sh, g0, g1, s0, s1) = scr
        gsem = (g0, g1)
        ssem = (s0, s1)

        c = lax.axis_index("c")
        s = lax.axis_index("s")
        base = s * ROWS_T
        vch = v_hbm.at[c]

        @pl.loop(0, RB)
        def _zr(r):
            for j in range(HC // L):
                rows_v[0, r, pl.ds(j * L, L)] = jnp.zeros((L,), jnp.float32)

        for k in range(5):
            pltpu.sync_copy(rows_v.at[0], acc_sh.at[pl.ds(base + k * RB, RB)])
        plsc.subcore_barrier()

        for ph in range(NPH):
            r0 = s * NB_ALL + ph * NBP
            pltpu.sync_copy(src_hbm.at[pl.ds(r0, NBP)], src_v)
            pltpu.sync_copy(dst_hbm.at[pl.ds(r0, NBP)], dst_v)
            pltpu.sync_copy(nrm_hbm.at[pl.ds(r0, NBP)], nrm_v)

            pltpu.async_copy(vch.at[src_v.at[0]], rows_v.at[0], gsem[0])

            @pl.loop(0, NBP // 2)
            def _grp(g):
                for p in range(2):
                    b = g * 2 + p

                    pltpu.make_async_copy(
                        vch.at[src_v.at[0]], rows_v.at[p], gsem[p]).wait()

                    @pl.when(b >= 1)
                    def _sw():
                        pltpu.make_async_copy(
                            rows_v.at[1 - p], acc_sh.at[dst_v.at[0]],
                            ssem[1 - p]).wait()

                    @pl.when(b + 1 < NBP)
                    def _gs():
                        pltpu.async_copy(
                            vch.at[src_v.at[b + 1]], rows_v.at[1 - p],
                            gsem[1 - p])

                    @pl.loop(0, EB // L)
                    def _sc(i16):
                        nv = nrm_v[b, pl.ds(i16 * L, L)]
                        for ii in range(L):
                            t = nv[ii]
                            i = i16 * L + ii
                            for j in range(HC // L):
                                sl = pl.ds(j * L, L)
                                rows_v[p, i, sl] = rows_v[p, i, sl] * t

                    pltpu.async_copy(
                        rows_v.at[p], acc_sh.at[dst_v.at[b]], ssem[p], add=True)

            pltpu.make_async_copy(
                rows_v.at[1], acc_sh.at[dst_v.at[0]], ssem[1]).wait()

        plsc.subcore_barrier()

        for k in range(5):
            rr = base + k * RB
            if with_sub:
                pltpu.sync_copy(acc_sh.at[pl.ds(rr, RB)], rows_v.at[0])
                pltpu.sync_copy(sub_hbm.at[c, pl.ds(rr, RB)], rows_v.at[1])

                @pl.loop(0, RB)
                def _cmb(r):
                    for j in range(HC // L):
                        sl = pl.ds(j * L, L)
                        rows_v[0, r, sl] = (rows_v[0, r, sl] * 2.0
                                            - rows_v[1, r, sl])

                pltpu.sync_copy(rows_v.at[0], out_hbm.at[c, pl.ds(rr, RB)])
            else:
                pltpu.sync_copy(acc_sh.at[pl.ds(rr, RB)],
                                out_hbm.at[c, pl.ds(rr, RB)])

    return functools.partial(
        pl.kernel,
        out_type=jax.ShapeDtypeStruct((NC, NPAD, HC), jnp.float32),
        mesh=_mesh,
        scratch_types=scratch,
        compiler_params=_sc_params,
    )(body)


_mv_plain = _make_mv(False)
_mv_sub = _make_mv(True)


RBLK = 1000


def _gate_body(x_ref, h_ref, c_ref, t1_ref, t2_ref, t3_ref, w_ref, b_ref,
               hn_ref, cn_ref):
    f32 = jnp.float32
    acc = jnp.dot(x_ref[...], w_ref[0:256, :], preferred_element_type=f32)
    acc = acc + jnp.dot(h_ref[...], w_ref[256:512, :], preferred_element_type=f32)
    off = 512
    for t_ref in (t1_ref, t2_ref, t3_ref):
        t = jnp.concatenate([t_ref[0], t_ref[1]], axis=1)
        acc = acc + jnp.dot(t, w_ref[off:off + 256, :],
                            preferred_element_type=f32)
        off += 256
    acc = acc + b_ref[...]
    gi = jax.nn.sigmoid(acc[:, 0:256])
    gf = jax.nn.sigmoid(acc[:, 256:512])
    gt = jnp.tanh(acc[:, 512:768])
    go = jax.nn.sigmoid(acc[:, 768:1024])
    cn = gf * c_ref[...] + gi * gt
    cn_ref[...] = cn
    hn_ref[...] = go * jnp.tanh(cn)


_gate_call = pl.pallas_call(
    _gate_body,
    grid=(N // RBLK,),
    in_specs=[
        pl.BlockSpec((RBLK, D), lambda i: (i, 0)),
        pl.BlockSpec((RBLK, D), lambda i: (i, 0)),
        pl.BlockSpec((RBLK, D), lambda i: (i, 0)),
        pl.BlockSpec((NC, RBLK, HC), lambda i: (0, i, 0)),
        pl.BlockSpec((NC, RBLK, HC), lambda i: (0, i, 0)),
        pl.BlockSpec((NC, RBLK, HC), lambda i: (0, i, 0)),
        pl.BlockSpec((5 * D, 4 * D), lambda i: (0, 0)),
        pl.BlockSpec((1, 4 * D), lambda i: (0, 0)),
    ],
    out_specs=[
        pl.BlockSpec((RBLK, D), lambda i: (i, 0)),
        pl.BlockSpec((RBLK, D), lambda i: (i, 0)),
    ],
    out_shape=[
        jax.ShapeDtypeStruct((N, D), jnp.float32),
        jax.ShapeDtypeStruct((N, D), jnp.float32),
    ],
)


def kernel(X, edge_index, edge_weight, H, C,
           W_i, b_i, W_f, b_f, W_c, b_c, W_o, b_o,
           Wc_i, bc_i, Wc_f, bc_f, Wc_c, bc_c, Wc_o, bc_o):
    f32 = jnp.float32
    src = edge_index[0]
    dst = edge_index[1]
    pad = EP - E
    srcp = jnp.concatenate([src, jnp.zeros((pad,), jnp.int32)]).reshape(ER, EB)
    dstp = jnp.concatenate([dst, jnp.zeros((pad,), jnp.int32)]).reshape(ER, EB)
    ewp = jnp.concatenate([edge_weight, jnp.zeros((pad,), f32)]).reshape(ER, EB)

    deg2 = _deg_call(srcp, ewp)
    dis = _dis_call(deg2.reshape(NC, NP // 128, 128))
    nrm = _norm_call(srcp, dstp, ewp, dis.reshape(NP))

    Hpad = jnp.concatenate([H, jnp.zeros((NPAD - N, D), f32)])
    Hc = Hpad.reshape(NPAD, NC, HC).transpose(1, 0, 2)
    Tx1 = _mv_plain(Hc, srcp, dstp, nrm)
    Tx2 = _mv_sub(Tx1, srcp, dstp, nrm, Hc)
    Tx3 = _mv_sub(Tx2, srcp, dstp, nrm, Tx1)

    Wbig = jnp.concatenate([
        jnp.concatenate([W_i, W_f, W_c, W_o], axis=1),
        jnp.concatenate([Wc_i[0], Wc_f[0], Wc_c[0], Wc_o[0]], axis=1),
        jnp.concatenate([Wc_i[1], Wc_f[1], Wc_c[1], Wc_o[1]], axis=1),
        jnp.concatenate([Wc_i[2], Wc_f[2], Wc_c[2], Wc_o[2]], axis=1),
        jnp.concatenate([Wc_i[3], Wc_f[3], Wc_c[3], Wc_o[3]], axis=1),
    ], axis=0)
    bias = jnp.concatenate([
        b_i + bc_i[None, :], b_f + bc_f[None, :],
        b_c + bc_c[None, :], b_o + bc_o[None, :],
    ], axis=1)

    Hn, Cn = _gate_call(X, H, C, Tx1, Tx2, Tx3, Wbig, bias)
    return (Hn, Cn)


# speedup vs baseline: 5.1681x; 1.0404x over previous
import functools

import jax
import jax.numpy as jnp
from jax import lax
from jax.experimental import pallas as pl
from jax.experimental.pallas import tpu as pltpu
from jax.experimental.pallas import tpu_sc as plsc

N = 10000
E = 160000
D = 256
NC, NS, L = 2, 16, 16
HC = D // NC
NPH = 2
NBP = 40
EB = 128
EP = 163840
ER = EP // EB
NB_ALL = EP // NS // EB
NB_HALF = EP // (NC * NS) // EB
NP = 10240
NPAD = 10240
ROWS_T = NPAD // NS
RB = 128

_mesh = plsc.VectorSubcoreMesh(
    core_axis_name="c", subcore_axis_name="s", num_cores=NC, num_subcores=NS)
_sc_params = pltpu.CompilerParams(needs_layout_passes=False)


@functools.partial(
    pl.kernel,
    out_type=jax.ShapeDtypeStruct((ER, EB), jnp.float32),
    mesh=_mesh,
    scratch_types=[
        pltpu.VMEM((NB_HALF, EB), jnp.int32),
        pltpu.VMEM((NB_HALF, EB), jnp.int32),
        pltpu.VMEM((NB_HALF, EB), jnp.float32),
        pltpu.VMEM((NB_HALF, EB), jnp.float32),
        pltpu.VMEM((NP,), jnp.float32),
        pltpu.VMEM((NP // NS,), jnp.float32),
        pltpu.VMEM_SHARED((NP,), jnp.float32),
        pltpu.VMEM_SHARED((NP,), jnp.float32),
    ],
    compiler_params=_sc_params,
)
def _prep_call(src_hbm, dst_hbm, ew_hbm, out_hbm,
               src_v, dst_v, ew_v, nrm_v, dis_v, seg_v, deg_sh, dis_sh):
    c = lax.axis_index("c")
    s = lax.axis_index("s")
    wid = c * NS + s
    seg = NP // NS

    @pl.loop(0, seg // L)
    def _zero(i):
        seg_v[pl.ds(i * L, L)] = jnp.zeros((L,), jnp.float32)

    pltpu.sync_copy(seg_v, deg_sh.at[pl.ds(s * seg, seg)])
    plsc.subcore_barrier()

    for ph in range(2):
        r0 = s * (2 * NB_HALF) + ph * NB_HALF
        pltpu.sync_copy(src_hbm.at[pl.ds(r0, NB_HALF)], src_v)
        pltpu.sync_copy(ew_hbm.at[pl.ds(r0, NB_HALF)], ew_v)

        @pl.loop(0, NB_HALF)
        def _scat(b):
            pltpu.sync_copy(ew_v.at[b], deg_sh.at[src_v.at[b]], add=True)

    plsc.subcore_barrier()

    pltpu.sync_copy(deg_sh.at[pl.ds(s * seg, seg)], seg_v)

    @pl.loop(0, seg // L)
    def _nr(i):
        sl = pl.ds(i * L, L)
        d = seg_v[sl]
        bits = plsc.bitcast(d, jnp.int32)
        y = plsc.bitcast(0x5F3759DF - lax.shift_right_logical(bits, 1),
                         jnp.float32)
        h = d * (-0.5)
        for _ in range(3):
            y = y * (1.5 + h * y * y)
        seg_v[sl] = jnp.where(d > 0.0, y, 0.0)

    pltpu.sync_copy(seg_v, dis_sh.at[pl.ds(s * seg, seg)])
    plsc.subcore_barrier()

    pltpu.sync_copy(dis_sh, dis_v)
    r0 = wid * NB_HALF
    pltpu.sync_copy(src_hbm.at[pl.ds(r0, NB_HALF)], src_v)
    pltpu.sync_copy(dst_hbm.at[pl.ds(r0, NB_HALF)], dst_v)
    pltpu.sync_copy(ew_hbm.at[pl.ds(r0, NB_HALF)], ew_v)

    @pl.loop(0, NB_HALF)
    def _row(b):
        for j in range(EB // L):
            sl = pl.ds(j * L, L)
            gs = plsc.load_gather(dis_v, [src_v[b, sl]])
            gd = plsc.load_gather(dis_v, [dst_v[b, sl]])
            nrm_v[b, sl] = -(ew_v[b, sl] * gs * gd)

    pltpu.sync_copy(nrm_v, out_hbm.at[pl.ds(r0, NB_HALF)])


def _make_mv(with_sub):
    scratch = [
        pltpu.VMEM((NBP, EB), jnp.int32),
        pltpu.VMEM((NBP, EB), jnp.int32),
        pltpu.VMEM((NBP, EB), jnp.float32),
        pltpu.VMEM((2, EB, HC), jnp.float32),
        pltpu.VMEM_SHARED((NPAD, HC), jnp.float32),
    ] + [pltpu.SemaphoreType.DMA] * 4

    def body(v_hbm, src_hbm, dst_hbm, nrm_hbm, *rest):
        if with_sub:
            sub_hbm, out_hbm = rest[0], rest[1]
            scr = rest[2:]
        else:
            sub_hbm, out_hbm = None, rest[0]
            scr = rest[1:]
        (src_v, dst_v, nrm_v, rows_v, acc_sh, g0, g1, s0, s1) = scr
        gsem = (g0, g1)
        ssem = (s0, s1)

        c = lax.axis_index("c")
        s = lax.axis_index("s")
        base = s * ROWS_T
        vch = v_hbm.at[c]

        @pl.loop(0, RB)
        def _zr(r):
            for j in range(HC // L):
                rows_v[0, r, pl.ds(j * L, L)] = jnp.zeros((L,), jnp.float32)

        for k in range(5):
            pltpu.sync_copy(rows_v.at[0], acc_sh.at[pl.ds(base + k * RB, RB)])
        plsc.subcore_barrier()

        for ph in range(NPH):
            r0 = s * NB_ALL + ph * NBP
            pltpu.sync_copy(src_hbm.at[pl.ds(r0, NBP)], src_v)
            pltpu.sync_copy(dst_hbm.at[pl.ds(r0, NBP)], dst_v)
            pltpu.sync_copy(nrm_hbm.at[pl.ds(r0, NBP)], nrm_v)

            pltpu.async_copy(vch.at[src_v.at[0]], rows_v.at[0], gsem[0])

            @pl.loop(0, NBP // 2)
            def _grp(g):
                for p in range(2):
                    b = g * 2 + p

                    pltpu.make_async_copy(
                        vch.at[src_v.at[0]], rows_v.at[p], gsem[p]).wait()

                    @pl.when(b >= 1)
                    def _sw():
                        pltpu.make_async_copy(
                            rows_v.at[1 - p], acc_sh.at[dst_v.at[0]],
                            ssem[1 - p]).wait()

                    @pl.when(b + 1 < NBP)
                    def _gs():
                        pltpu.async_copy(
                            vch.at[src_v.at[b + 1]], rows_v.at[1 - p],
                            gsem[1 - p])

                    @pl.loop(0, EB // L)
                    def _sc(i16):
                        nv = nrm_v[b, pl.ds(i16 * L, L)]
                        for ii in range(L):
                            t = nv[ii]
                            i = i16 * L + ii
                            for j in range(HC // L):
                                sl = pl.ds(j * L, L)
                                rows_v[p, i, sl] = rows_v[p, i, sl] * t

                    pltpu.async_copy(
                        rows_v.at[p], acc_sh.at[dst_v.at[b]], ssem[p], add=True)

            pltpu.make_async_copy(
                rows_v.at[1], acc_sh.at[dst_v.at[0]], ssem[1]).wait()

        plsc.subcore_barrier()

        for k in range(5):
            rr = base + k * RB
            if with_sub:
                pltpu.sync_copy(acc_sh.at[pl.ds(rr, RB)], rows_v.at[0])
                pltpu.sync_copy(sub_hbm.at[c, pl.ds(rr, RB)], rows_v.at[1])

                @pl.loop(0, RB)
                def _cmb(r):
                    for j in range(HC // L):
                        sl = pl.ds(j * L, L)
                        rows_v[0, r, sl] = (rows_v[0, r, sl] * 2.0
                                            - rows_v[1, r, sl])

                pltpu.sync_copy(rows_v.at[0], out_hbm.at[c, pl.ds(rr, RB)])
            else:
                pltpu.sync_copy(acc_sh.at[pl.ds(rr, RB)],
                                out_hbm.at[c, pl.ds(rr, RB)])

    return functools.partial(
        pl.kernel,
        out_type=jax.ShapeDtypeStruct((NC, NPAD, HC), jnp.float32),
        mesh=_mesh,
        scratch_types=scratch,
        compiler_params=_sc_params,
    )(body)


_mv_plain = _make_mv(False)
_mv_sub = _make_mv(True)


RBLK = 1000


def _gate_body(x_ref, h_ref, c_ref, t1_ref, t2_ref, t3_ref, w_ref, b_ref,
               hn_ref, cn_ref):
    f32 = jnp.float32
    acc = jnp.dot(x_ref[...], w_ref[0:256, :], preferred_element_type=f32)
    acc = acc + jnp.dot(h_ref[...], w_ref[256:512, :], preferred_element_type=f32)
    off = 512
    for t_ref in (t1_ref, t2_ref, t3_ref):
        t = jnp.concatenate([t_ref[0], t_ref[1]], axis=1)
        acc = acc + jnp.dot(t, w_ref[off:off + 256, :],
                            preferred_element_type=f32)
        off += 256
    acc = acc + b_ref[...]
    gi = jax.nn.sigmoid(acc[:, 0:256])
    gf = jax.nn.sigmoid(acc[:, 256:512])
    gt = jnp.tanh(acc[:, 512:768])
    go = jax.nn.sigmoid(acc[:, 768:1024])
    cn = gf * c_ref[...] + gi * gt
    cn_ref[...] = cn
    hn_ref[...] = go * jnp.tanh(cn)


_gate_call = pl.pallas_call(
    _gate_body,
    grid=(N // RBLK,),
    in_specs=[
        pl.BlockSpec((RBLK, D), lambda i: (i, 0)),
        pl.BlockSpec((RBLK, D), lambda i: (i, 0)),
        pl.BlockSpec((RBLK, D), lambda i: (i, 0)),
        pl.BlockSpec((NC, RBLK, HC), lambda i: (0, i, 0)),
        pl.BlockSpec((NC, RBLK, HC), lambda i: (0, i, 0)),
        pl.BlockSpec((NC, RBLK, HC), lambda i: (0, i, 0)),
        pl.BlockSpec((5 * D, 4 * D), lambda i: (0, 0)),
        pl.BlockSpec((1, 4 * D), lambda i: (0, 0)),
    ],
    out_specs=[
        pl.BlockSpec((RBLK, D), lambda i: (i, 0)),
        pl.BlockSpec((RBLK, D), lambda i: (i, 0)),
    ],
    out_shape=[
        jax.ShapeDtypeStruct((N, D), jnp.float32),
        jax.ShapeDtypeStruct((N, D), jnp.float32),
    ],
)


def kernel(X, edge_index, edge_weight, H, C,
           W_i, b_i, W_f, b_f, W_c, b_c, W_o, b_o,
           Wc_i, bc_i, Wc_f, bc_f, Wc_c, bc_c, Wc_o, bc_o):
    f32 = jnp.float32
    src = edge_index[0]
    dst = edge_index[1]
    pad = EP - E
    srcp = jnp.concatenate([src, jnp.zeros((pad,), jnp.int32)]).reshape(ER, EB)
    dstp = jnp.concatenate([dst, jnp.zeros((pad,), jnp.int32)]).reshape(ER, EB)
    ewp = jnp.concatenate([edge_weight, jnp.zeros((pad,), f32)]).reshape(ER, EB)

    nrm = _prep_call(srcp, dstp, ewp)

    Hpad = jnp.concatenate([H, jnp.zeros((NPAD - N, D), f32)])
    Hc = Hpad.reshape(NPAD, NC, HC).transpose(1, 0, 2)
    Tx1 = _mv_plain(Hc, srcp, dstp, nrm)
    Tx2 = _mv_sub(Tx1, srcp, dstp, nrm, Hc)
    Tx3 = _mv_sub(Tx2, srcp, dstp, nrm, Tx1)

    Wbig = jnp.concatenate([
        jnp.concatenate([W_i, W_f, W_c, W_o], axis=1),
        jnp.concatenate([Wc_i[0], Wc_f[0], Wc_c[0], Wc_o[0]], axis=1),
        jnp.concatenate([Wc_i[1], Wc_f[1], Wc_c[1], Wc_o[1]], axis=1),
        jnp.concatenate([Wc_i[2], Wc_f[2], Wc_c[2], Wc_o[2]], axis=1),
        jnp.concatenate([Wc_i[3], Wc_f[3], Wc_c[3], Wc_o[3]], axis=1),
    ], axis=0)
    bias = jnp.concatenate([
        b_i + bc_i[None, :], b_f + bc_f[None, :],
        b_c + bc_c[None, :], b_o + bc_o[None, :],
    ], axis=1)

    Hn, Cn = _gate_call(X, H, C, Tx1, Tx2, Tx3, Wbig, bias)
    return (Hn, Cn)
